# baseline (speedup 1.0000x reference)
import functools

import jax
import jax.numpy as jnp
from jax import lax
from jax.experimental import pallas as pl
from jax.experimental.pallas import tpu as pltpu
from jax.experimental.pallas import tpu_sc as plsc

F = 26
V = 100000
E = 16
B = 16384
NP = F * (F - 1) // 2
NIN = F * E

NC = 2
NS = 16
NW = NC * NS
ROWS_PER_W = NIN // NW
IC = 4096
NCK = B // IC
H0 = 49920
H1 = 50048
TAILW = 128
TBASE = V - TAILW
TROWS = 24


def _sc_gather(table_h, tail_h, idxt_h, out_h, pa, pb, tailv, ia, ib, out_v,
               pasem, pbsem, tsem, isem, osem):
    wid = lax.axis_index("s") * NC + lax.axis_index("c")
    r0 = wid * ROWS_PER_W
    f0 = r0 // E

    pltpu.async_copy(table_h.at[r0, pl.ds(0, H0)], pa, pasem)
    pltpu.async_copy(table_h.at[r0, pl.ds(H0, H1)], pb.at[pl.ds(0, H1)], pbsem)
    r0a = jnp.minimum((r0 // 8) * 8, NIN - TROWS)
    du = r0 - r0a
    pltpu.async_copy(tail_h.at[pl.ds(r0a, TROWS), :], tailv, tsem)
    pltpu.async_copy(idxt_h.at[f0, pl.ds(0, IC)], ia, isem)
    for c in range(NCK):
        pltpu.async_copy(table_h.at[r0, pl.ds(0, IC)],
                         out_v.at[pl.ds(c * IC, IC)], osem)
    pltpu.make_async_copy(tail_h.at[pl.ds(r0a, TROWS), :], tailv, tsem).wait()
    iota = jax.lax.iota(jnp.int32, E)

    def unit(u, carry):
        r = r0 + u
        f = r // E
        rn = jnp.minimum(r + 1, NIN - 1)
        fn = rn // E
        not_last = u < ROWS_PER_W - 1

        for c in range(NCK):
            pltpu.make_async_copy(out_v.at[pl.ds(c * IC, IC)],
                                  out_h.at[r, pl.ds(c * IC, IC)], osem).wait()

        pltpu.make_async_copy(table_h.at[r, pl.ds(0, H0)], pa, pasem).wait()
        for c in range(NCK):
            ibuf = ia if c % 2 == 0 else ib
            nbuf = ib if c % 2 == 0 else ia
            pltpu.make_async_copy(idxt_h.at[f, pl.ds(0, IC)], ibuf,
                                  isem).wait()
            if c < NCK - 1:
                pltpu.async_copy(idxt_h.at[f, pl.ds((c + 1) * IC, IC)],
                                 nbuf, isem)
            else:
                pltpu.async_copy(idxt_h.at[f, pl.ds(0, IC)], nbuf, isem)

            @plsc.parallel_loop(0, IC // E, unroll=16)
            def _(k):
                iv = ibuf[pl.ds(k * E, E)]
                loc = jnp.minimum(iv, H0 - 1)
                out_v[pl.ds(c * IC + k * E, E)] = plsc.load_gather(pa, [loc])

        @pl.when(not_last)
        def _():
            pltpu.async_copy(table_h.at[rn, pl.ds(0, H0)], pa, pasem)

        pltpu.make_async_copy(table_h.at[r, pl.ds(H0, H1)],
                              pb.at[pl.ds(0, H1)], pbsem).wait()
        for t in range(2):
            pb[pl.ds(H1 + t * E, E)] = tailv[du + u, pl.ds(
                TAILW - 2 * E + t * E, E)]
        for c in range(NCK):
            ibuf = ia if c % 2 == 0 else ib
            nbuf = ib if c % 2 == 0 else ia
            pltpu.make_async_copy(idxt_h.at[f, pl.ds(0, IC)], ibuf,
                                  isem).wait()

            @pl.when(jnp.logical_or(c < NCK - 1, not_last))
            def _():
                if c < NCK - 1:
                    pltpu.async_copy(idxt_h.at[f, pl.ds((c + 1) * IC, IC)],
                                     nbuf, isem)
                else:
                    pltpu.async_copy(idxt_h.at[fn, pl.ds(0, IC)], nbuf, isem)

            @plsc.parallel_loop(0, IC // E, unroll=16)
            def _(k):
                base = c * IC + k * E
                iv = ibuf[pl.ds(k * E, E)]
                m = iv >= H0
                loc = jnp.clip(iv - H0, 0, H1 + 2 * E - 1)
                g = plsc.load_gather(pb, [loc])
                plsc.store_scatter(out_v, [base + iota], g, mask=m)

            pltpu.async_copy(out_v.at[pl.ds(c * IC, IC)],
                             out_h.at[r, pl.ds(c * IC, IC)], osem)

        @pl.when(not_last)
        def _():
            pltpu.async_copy(table_h.at[rn, pl.ds(H0, H1)],
                             pb.at[pl.ds(0, H1)], pbsem)

        return carry

    lax.fori_loop(0, ROWS_PER_W, unit, 0)
    rl = r0 + ROWS_PER_W - 1
    for c in range(NCK):
        pltpu.make_async_copy(out_v.at[pl.ds(c * IC, IC)],
                              out_h.at[rl, pl.ds(c * IC, IC)], osem).wait()


def _gather_call(table_t, table_tail, idx_t):
    mesh = plsc.VectorSubcoreMesh(core_axis_name="c", subcore_axis_name="s")
    return pl.kernel(
        _sc_gather,
        mesh=mesh,
        out_type=jax.ShapeDtypeStruct((NIN, B), jnp.float32),
        scratch_types=[
            pltpu.VMEM((H0,), jnp.float32),
            pltpu.VMEM((H1 + 2 * E,), jnp.float32),
            pltpu.VMEM((TROWS, TAILW), jnp.float32),
            pltpu.VMEM((IC,), jnp.int32),
            pltpu.VMEM((IC,), jnp.int32),
            pltpu.VMEM((B,), jnp.float32),
            pltpu.SemaphoreType.DMA,
            pltpu.SemaphoreType.DMA,
            pltpu.SemaphoreType.DMA,
            pltpu.SemaphoreType.DMA,
            pltpu.SemaphoreType.DMA,
        ],
        compiler_params=pltpu.CompilerParams(
            use_tc_tiling_on_sc=True, needs_layout_passes=False
        ),
    )(table_t, table_tail, idx_t)


BB = 2048


def _tc_body(x_ref, w0at_ref, w0bt8_ref, b0_ref, w1_ref, b1_ref, o_ref, ipt_ref):
    xt = x_ref[:]
    xb = xt.astype(jnp.bfloat16)
    off = 0
    for i in range(F - 1):
        n = F - 1 - i
        a = xt[i * E:(i + 1) * E, :]
        rest = xt[(i + 1) * E:, :]
        prod = rest.reshape(n, E, BB) * a[None, :, :]
        s8 = prod[:, :8, :] + prod[:, 8:, :]
        ipt_ref[pl.ds(off * 8, n * 8), :] = s8.reshape(n * 8, BB).astype(
            jnp.bfloat16)
        off += n
    ht = (
        jnp.dot(w0at_ref[:], xb, preferred_element_type=jnp.float32)
        + jnp.dot(w0bt8_ref[:], ipt_ref[:], preferred_element_type=jnp.float32)
        + b0_ref[:]
    )
    ht = jnp.maximum(ht, 0.0)
    logit = jnp.dot(w1_ref[:], ht, preferred_element_type=jnp.float32) + b1_ref[:]
    o_ref[:] = (1.0 / (1.0 + jnp.exp(-logit))).reshape(1, 1, BB)


def _mlp_call(xt, w0at, w0bt8, b0c, w1r, b1c):
    grid = (B // BB,)
    return pl.pallas_call(
        _tc_body,
        grid=grid,
        in_specs=[
            pl.BlockSpec((NIN, BB), lambda i: (0, i)),
            pl.BlockSpec((400, NIN), lambda i: (0, 0)),
            pl.BlockSpec((400, NP * 8), lambda i: (0, 0)),
            pl.BlockSpec((400, 1), lambda i: (0, 0)),
            pl.BlockSpec((1, 400), lambda i: (0, 0)),
            pl.BlockSpec((1, 1), lambda i: (0, 0)),
        ],
        out_specs=pl.BlockSpec((1, 1, BB), lambda i: (i, 0, 0)),
        out_shape=jax.ShapeDtypeStruct((B // BB, 1, BB), jnp.float32),
        scratch_shapes=[pltpu.VMEM((NP * 8, BB), jnp.bfloat16)],
    )(xt, w0at, w0bt8, b0c, w1r, b1c)


def kernel(indices, tables, w0, b0, w1, b1):
    table_t = jnp.transpose(tables, (0, 2, 1)).reshape(NIN, V)
    table_tail = table_t[:, TBASE:]
    idx_t = indices.T
    xt = _gather_call(table_t, table_tail, idx_t)
    w0at = w0[:NIN].T.astype(jnp.bfloat16)
    w0bt8 = jnp.repeat(w0[NIN:].T, 8, axis=1).astype(jnp.bfloat16)
    y2d = _mlp_call(xt, w0at, w0bt8, b0.reshape(400, 1), w1.reshape(1, 400),
                    b1.reshape(1, 1))
    return y2d.reshape(B)


# speedup vs baseline: 1.0992x; 1.0992x over previous
import functools

import jax
import jax.numpy as jnp
from jax import lax
from jax.experimental import pallas as pl
from jax.experimental.pallas import tpu as pltpu
from jax.experimental.pallas import tpu_sc as plsc

F = 26
V = 100000
E = 16
B = 16384
NP = F * (F - 1) // 2
NIN = F * E

NC = 2
NS = 16
NW = NC * NS
ROWS_PER_W = NIN // NW
IC = 4096
NCK = B // IC
H0 = 49920
H1 = 50048
TAILW = 128
TBASE = V - TAILW
TROWS = 24


def _stage_plane(table_h, plane, psem, r):
    pltpu.async_copy(table_h.at[r, pl.ds(0, H0)], plane.at[pl.ds(0, H0)], psem)
    pltpu.async_copy(table_h.at[r, pl.ds(H0, H1)], plane.at[pl.ds(H0, H1)],
                     psem)


def _wait_plane(table_h, plane, psem, r):
    pltpu.make_async_copy(table_h.at[r, pl.ds(0, H0)], plane.at[pl.ds(0, H0)],
                          psem).wait()
    pltpu.make_async_copy(table_h.at[r, pl.ds(H0, H1)],
                          plane.at[pl.ds(H0, H1)], psem).wait()


def _sc_gather(table_h, tail_h, idxt_h, out_h, plane, tailv, ia, ib, oa, ob,
               psem, tsem, isem, oasem, obsem):
    wid = lax.axis_index("s") * NC + lax.axis_index("c")
    r0 = wid * ROWS_PER_W
    f0 = r0 // E

    _stage_plane(table_h, plane, psem, r0)
    r0a = jnp.minimum((r0 // 8) * 8, NIN - TROWS)
    du = r0 - r0a
    pltpu.async_copy(tail_h.at[pl.ds(r0a, TROWS), :], tailv, tsem)
    pltpu.async_copy(idxt_h.at[f0, pl.ds(0, IC)], ia, isem)
    pltpu.async_copy(table_h.at[r0, pl.ds(0, IC)], oa, oasem)
    pltpu.async_copy(table_h.at[r0, pl.ds(0, IC)], ob, obsem)
    pltpu.make_async_copy(tail_h.at[pl.ds(r0a, TROWS), :], tailv, tsem).wait()

    def unit(u, carry):
        r = r0 + u
        f = r // E
        rn = jnp.minimum(r + 1, NIN - 1)
        fn = rn // E
        not_last = u < ROWS_PER_W - 1

        _wait_plane(table_h, plane, psem, r)
        for t in range(2):
            plane[pl.ds(H0 + H1 + t * E, E)] = tailv[du + u, pl.ds(
                TAILW - 2 * E + t * E, E)]
        for c in range(NCK):
            ibuf = ia if c % 2 == 0 else ib
            nbuf = ib if c % 2 == 0 else ia
            obuf = oa if c % 2 == 0 else ob
            osem = oasem if c % 2 == 0 else obsem
            pltpu.make_async_copy(idxt_h.at[f, pl.ds(0, IC)], ibuf,
                                  isem).wait()
            if c < NCK - 1:
                pltpu.async_copy(idxt_h.at[f, pl.ds((c + 1) * IC, IC)],
                                 nbuf, isem)
            else:
                @pl.when(not_last)
                def _():
                    pltpu.async_copy(idxt_h.at[fn, pl.ds(0, IC)], nbuf, isem)
            pltpu.make_async_copy(obuf, out_h.at[r, pl.ds(c * IC, IC)],
                                  osem).wait()

            @plsc.parallel_loop(0, IC // E, unroll=16)
            def _(k):
                iv = ibuf[pl.ds(k * E, E)]
                obuf[pl.ds(k * E, E)] = plsc.load_gather(plane, [iv])

            pltpu.async_copy(obuf, out_h.at[r, pl.ds(c * IC, IC)], osem)
        @pl.when(not_last)
        def _():
            _stage_plane(table_h, plane, psem, rn)

        return carry

    lax.fori_loop(0, ROWS_PER_W, unit, 0)
    rl = r0 + ROWS_PER_W - 1
    pltpu.make_async_copy(oa, out_h.at[rl, pl.ds(2 * IC, IC)], oasem).wait()
    pltpu.make_async_copy(ob, out_h.at[rl, pl.ds(3 * IC, IC)], obsem).wait()


def _gather_call(table_t, table_tail, idx_t):
    mesh = plsc.VectorSubcoreMesh(core_axis_name="c", subcore_axis_name="s")
    return pl.kernel(
        _sc_gather,
        mesh=mesh,
        out_type=jax.ShapeDtypeStruct((NIN, B), jnp.float32),
        scratch_types=[
            pltpu.VMEM((V,), jnp.float32),
            pltpu.VMEM((TROWS, TAILW), jnp.float32),
            pltpu.VMEM((IC,), jnp.int32),
            pltpu.VMEM((IC,), jnp.int32),
            pltpu.VMEM((IC,), jnp.float32),
            pltpu.VMEM((IC,), jnp.float32),
            pltpu.SemaphoreType.DMA,
            pltpu.SemaphoreType.DMA,
            pltpu.SemaphoreType.DMA,
            pltpu.SemaphoreType.DMA,
            pltpu.SemaphoreType.DMA,
        ],
        compiler_params=pltpu.CompilerParams(
            use_tc_tiling_on_sc=True, needs_layout_passes=False
        ),
    )(table_t, table_tail, idx_t)


BB = 2048


def _tc_body(x_ref, w0at_ref, w0bt8_ref, b0_ref, w1_ref, b1_ref, o_ref, ipt_ref):
    xt = x_ref[:]
    xb = xt.astype(jnp.bfloat16)
    off = 0
    for i in range(F - 1):
        n = F - 1 - i
        a = xt[i * E:(i + 1) * E, :]
        rest = xt[(i + 1) * E:, :]
        prod = rest.reshape(n, E, BB) * a[None, :, :]
        s8 = prod[:, :8, :] + prod[:, 8:, :]
        ipt_ref[pl.ds(off * 8, n * 8), :] = s8.reshape(n * 8, BB).astype(
            jnp.bfloat16)
        off += n
    ht = (
        jnp.dot(w0at_ref[:], xb, preferred_element_type=jnp.float32)
        + jnp.dot(w0bt8_ref[:], ipt_ref[:], preferred_element_type=jnp.float32)
        + b0_ref[:]
    )
    ht = jnp.maximum(ht, 0.0)
    logit = jnp.dot(w1_ref[:], ht, preferred_element_type=jnp.float32) + b1_ref[:]
    o_ref[:] = (1.0 / (1.0 + jnp.exp(-logit))).reshape(1, 1, BB)


def _mlp_call(xt, w0at, w0bt8, b0c, w1r, b1c):
    grid = (B // BB,)
    return pl.pallas_call(
        _tc_body,
        grid=grid,
        in_specs=[
            pl.BlockSpec((NIN, BB), lambda i: (0, i)),
            pl.BlockSpec((400, NIN), lambda i: (0, 0)),
            pl.BlockSpec((400, NP * 8), lambda i: (0, 0)),
            pl.BlockSpec((400, 1), lambda i: (0, 0)),
            pl.BlockSpec((1, 400), lambda i: (0, 0)),
            pl.BlockSpec((1, 1), lambda i: (0, 0)),
        ],
        out_specs=pl.BlockSpec((1, 1, BB), lambda i: (i, 0, 0)),
        out_shape=jax.ShapeDtypeStruct((B // BB, 1, BB), jnp.float32),
        scratch_shapes=[pltpu.VMEM((NP * 8, BB), jnp.bfloat16)],
    )(xt, w0at, w0bt8, b0c, w1r, b1c)


def kernel(indices, tables, w0, b0, w1, b1):
    table_t = jnp.transpose(tables, (0, 2, 1)).reshape(NIN, V)
    table_tail = table_t[:, TBASE:]
    idx_t = indices.T
    xt = _gather_call(table_t, table_tail, idx_t)
    w0at = w0[:NIN].T.astype(jnp.bfloat16)
    w0bt8 = jnp.repeat(w0[NIN:].T, 8, axis=1).astype(jnp.bfloat16)
    y2d = _mlp_call(xt, w0at, w0bt8, b0.reshape(400, 1), w1.reshape(1, 400),
                    b1.reshape(1, 1))
    return y2d.reshape(B)


# speedup vs baseline: 1.1009x; 1.0016x over previous
import jax
import jax.numpy as jnp
from jax import lax
from jax.experimental import pallas as pl
from jax.experimental.pallas import tpu as pltpu
from jax.experimental.pallas import tpu_sc as plsc

F = 26
V = 100000
E = 16
B = 16384
NP = F * (F - 1) // 2
NIN = F * E

NC = 2
NS = 16
NW = NC * NS
ROWS_PER_W = NIN // NW
IC = 4096
NCK = B // IC
H0 = 49920
H1 = 50048
TAILW = 128
TBASE = V - TAILW
TROWS = 24


def _stage_plane(table_h, plane, psem, r):
    pltpu.async_copy(table_h.at[r, pl.ds(0, H0)], plane.at[pl.ds(0, H0)], psem)
    pltpu.async_copy(table_h.at[r, pl.ds(H0, H1)], plane.at[pl.ds(H0, H1)],
                     psem)


def _wait_plane(table_h, plane, psem, r):
    pltpu.make_async_copy(table_h.at[r, pl.ds(0, H0)], plane.at[pl.ds(0, H0)],
                          psem).wait()
    pltpu.make_async_copy(table_h.at[r, pl.ds(H0, H1)],
                          plane.at[pl.ds(H0, H1)], psem).wait()


def _sc_gather(table_h, tail_h, idxt_h, out_h, plane, tailv, ia, ib, oa, ob,
               psem, tsem, isem, oasem, obsem):
    wid = lax.axis_index("s") * NC + lax.axis_index("c")
    r0 = wid * ROWS_PER_W
    f0 = r0 // E

    _stage_plane(table_h, plane, psem, r0)
    r0a = jnp.minimum((r0 // 8) * 8, NIN - TROWS)
    du = r0 - r0a
    pltpu.async_copy(tail_h.at[pl.ds(r0a, TROWS), :], tailv, tsem)
    pltpu.async_copy(idxt_h.at[f0, pl.ds(0, IC)], ia, isem)
    pltpu.async_copy(table_h.at[r0, pl.ds(0, IC)], oa, oasem)
    pltpu.async_copy(table_h.at[r0, pl.ds(0, IC)], ob, obsem)
    pltpu.make_async_copy(tail_h.at[pl.ds(r0a, TROWS), :], tailv, tsem).wait()

    def unit(u, carry):
        r = r0 + u
        f = r // E
        rn = jnp.minimum(r + 1, NIN - 1)
        fn = rn // E
        not_last = u < ROWS_PER_W - 1

        _wait_plane(table_h, plane, psem, r)
        for t in range(2):
            plane[pl.ds(H0 + H1 + t * E, E)] = tailv[du + u, pl.ds(
                TAILW - 2 * E + t * E, E)]
        for c in range(NCK):
            ibuf = ia if c % 2 == 0 else ib
            nbuf = ib if c % 2 == 0 else ia
            obuf = oa if c % 2 == 0 else ob
            osem = oasem if c % 2 == 0 else obsem
            pltpu.make_async_copy(idxt_h.at[f, pl.ds(0, IC)], ibuf,
                                  isem).wait()
            if c < NCK - 1:
                pltpu.async_copy(idxt_h.at[f, pl.ds((c + 1) * IC, IC)],
                                 nbuf, isem)
            else:
                @pl.when(not_last)
                def _():
                    pltpu.async_copy(idxt_h.at[fn, pl.ds(0, IC)], nbuf, isem)
            pltpu.make_async_copy(obuf, out_h.at[r, pl.ds(c * IC, IC)],
                                  osem).wait()

            @plsc.parallel_loop(0, IC // E, unroll=16)
            def _(k):
                iv = ibuf[pl.ds(k * E, E)]
                obuf[pl.ds(k * E, E)] = plsc.load_gather(plane, [iv])

            pltpu.async_copy(obuf, out_h.at[r, pl.ds(c * IC, IC)], osem)
        @pl.when(not_last)
        def _():
            _stage_plane(table_h, plane, psem, rn)

        return carry

    lax.fori_loop(0, ROWS_PER_W, unit, 0)
    rl = r0 + ROWS_PER_W - 1
    pltpu.make_async_copy(oa, out_h.at[rl, pl.ds(2 * IC, IC)], oasem).wait()
    pltpu.make_async_copy(ob, out_h.at[rl, pl.ds(3 * IC, IC)], obsem).wait()


def _gather_call(table_t, table_tail, idx_t):
    mesh = plsc.VectorSubcoreMesh(core_axis_name="c", subcore_axis_name="s")
    return pl.kernel(
        _sc_gather,
        mesh=mesh,
        out_type=jax.ShapeDtypeStruct((NIN, B), jnp.float32),
        scratch_types=[
            pltpu.VMEM((V,), jnp.float32),
            pltpu.VMEM((TROWS, TAILW), jnp.float32),
            pltpu.VMEM((IC,), jnp.int32),
            pltpu.VMEM((IC,), jnp.int32),
            pltpu.VMEM((IC,), jnp.float32),
            pltpu.VMEM((IC,), jnp.float32),
            pltpu.SemaphoreType.DMA,
            pltpu.SemaphoreType.DMA,
            pltpu.SemaphoreType.DMA,
            pltpu.SemaphoreType.DMA,
            pltpu.SemaphoreType.DMA,
        ],
        compiler_params=pltpu.CompilerParams(
            use_tc_tiling_on_sc=True, needs_layout_passes=False
        ),
    )(table_t, table_tail, idx_t)


BB = 2048


def _tc_body(x_ref, w0at_ref, w0bt8_ref, b0_ref, w1_ref, b1_ref, o_ref, ipt_ref):
    xt = x_ref[:]
    xb = xt.astype(jnp.bfloat16)
    off = 0
    for i in range(F - 1):
        n = F - 1 - i
        a = xt[i * E:(i + 1) * E, :]
        rest = xt[(i + 1) * E:, :]
        prod = rest.reshape(n, E, BB) * a[None, :, :]
        s8 = prod[:, :8, :] + prod[:, 8:, :]
        ipt_ref[pl.ds(off * 8, n * 8), :] = s8.reshape(n * 8, BB).astype(
            jnp.bfloat16)
        off += n
    ht = (
        jnp.dot(w0at_ref[:], xb, preferred_element_type=jnp.float32)
        + jnp.dot(w0bt8_ref[:], ipt_ref[:], preferred_element_type=jnp.float32)
        + b0_ref[:]
    )
    ht = jnp.maximum(ht, 0.0)
    logit = jnp.dot(w1_ref[:], ht, preferred_element_type=jnp.float32) + b1_ref[:]
    o_ref[:] = (1.0 / (1.0 + jnp.exp(-logit))).reshape(1, 1, BB)


def _mlp_call(xt, w0at, w0bt8, b0c, w1r, b1c):
    grid = (B // BB,)
    return pl.pallas_call(
        _tc_body,
        grid=grid,
        in_specs=[
            pl.BlockSpec((NIN, BB), lambda i: (0, i)),
            pl.BlockSpec((400, NIN), lambda i: (0, 0)),
            pl.BlockSpec((400, NP * 8), lambda i: (0, 0)),
            pl.BlockSpec((400, 1), lambda i: (0, 0)),
            pl.BlockSpec((1, 400), lambda i: (0, 0)),
            pl.BlockSpec((1, 1), lambda i: (0, 0)),
        ],
        out_specs=pl.BlockSpec((1, 1, BB), lambda i: (i, 0, 0)),
        out_shape=jax.ShapeDtypeStruct((B // BB, 1, BB), jnp.float32),
        scratch_shapes=[pltpu.VMEM((NP * 8, BB), jnp.bfloat16)],
    )(xt, w0at, w0bt8, b0c, w1r, b1c)


def kernel(indices, tables, w0, b0, w1, b1):
    table_t = jnp.transpose(tables, (0, 2, 1)).reshape(NIN, V)
    table_tail = table_t[:, TBASE:]
    idx_t = indices.T
    xt = _gather_call(table_t, table_tail, idx_t)
    w0at = w0[:NIN].T.astype(jnp.bfloat16)
    w0bt8 = jnp.repeat(w0[NIN:].T, 8, axis=1).astype(jnp.bfloat16)
    y2d = _mlp_call(xt, w0at, w0bt8, b0.reshape(400, 1), w1.reshape(1, 400),
                    b1.reshape(1, 1))
    return y2d.reshape(B)
